# baseline (speedup 1.0000x reference)
import functools
import jax
import jax.numpy as jnp
from jax import lax
from jax.experimental import pallas as pl
from jax.experimental.pallas import tpu as pltpu
from jax.experimental.pallas import tpu_sc as plsc

N = 100000
D = 256
NC = 2
NS = 16
NW = NC * NS
ROWS_W = -(-(N // 8) // NW) * 8
C = 120
FULL_CHUNKS = ROWS_W // C
TAIL = ROWS_W - FULL_CHUNKS * C
NCH = FULL_CHUNKS + (1 if TAIL else 0)

_mesh = plsc.VectorSubcoreMesh(core_axis_name="c", subcore_axis_name="s")


@functools.partial(
    pl.kernel,
    out_type=jax.ShapeDtypeStruct((N, 2 * D), jnp.float32),
    mesh=_mesh,
    scratch_types=[
        pltpu.VMEM((2, C, 2 * D), jnp.float32),
        pltpu.SemaphoreType.DMA((2,)),
        pltpu.SemaphoreType.DMA((2,)),
    ],
)
def _sc_concat(s_hbm, d_hbm, o_hbm, buf, g_sem, s_sem):
    wid = lax.axis_index("s") * NC + lax.axis_index("c")
    base = jnp.minimum(wid * ROWS_W, N - ROWS_W)
    base = pl.multiple_of(base, 8)

    def sizes(i):
        return (i * C, C if i < FULL_CHUNKS else TAIL)

    def gathers(i, b):
        off, sz = sizes(i)
        rows = pl.ds(base + off, sz)
        return (
            pltpu.make_async_copy(
                s_hbm.at[rows, :], buf.at[b, pl.ds(0, sz), pl.ds(0, D)], g_sem.at[b]
            ),
            pltpu.make_async_copy(
                d_hbm.at[rows, :], buf.at[b, pl.ds(0, sz), pl.ds(D, D)], g_sem.at[b]
            ),
        )

    def scatter(i, b):
        off, sz = sizes(i)
        rows = pl.ds(base + off, sz)
        return pltpu.make_async_copy(
            buf.at[b, pl.ds(0, sz), :], o_hbm.at[rows, :], s_sem.at[b]
        )

    for i in range(NCH + 1):
        b = i % 2
        pb = (i - 1) % 2
        if i < NCH:
            if i >= 2:
                scatter(i - 2, b).wait()
            g1, g2 = gathers(i, b)
            g1.start()
            g2.start()
        if i >= 1:
            g1, g2 = gathers(i - 1, pb)
            g1.wait()
            g2.wait()
            scatter(i - 1, pb).start()
    for i in (NCH - 2, NCH - 1):
        scatter(i, i % 2).wait()


def kernel(static_emb, dynamic_emb):
    return _sc_concat(static_emb, dynamic_emb)


# speedup vs baseline: 1.0106x; 1.0106x over previous
import functools
import jax
import jax.numpy as jnp
from jax import lax
from jax.experimental import pallas as pl
from jax.experimental.pallas import tpu as pltpu
from jax.experimental.pallas import tpu_sc as plsc

N = 100000
D = 256
NC = 2
NS = 16
NW = NC * NS
ROWS_W = -(-(N // 8) // NW) * 8
C = 240
FULL_CHUNKS = ROWS_W // C
TAIL = ROWS_W - FULL_CHUNKS * C
NCH = FULL_CHUNKS + (1 if TAIL else 0)

_ITEMS = [(c, w) for c in range(NCH) for w in (0, 1)]

_mesh = plsc.VectorSubcoreMesh(core_axis_name="c", subcore_axis_name="s")


@functools.partial(
    pl.kernel,
    out_type=jax.ShapeDtypeStruct((N, 2 * D), jnp.float32),
    mesh=_mesh,
    scratch_types=[
        pltpu.VMEM((2, C, D), jnp.float32),
        pltpu.SemaphoreType.DMA((2,)),
        pltpu.SemaphoreType.DMA((2,)),
    ],
)
def _sc_concat(s_hbm, d_hbm, o_hbm, buf, g_sem, s_sem):
    wid = lax.axis_index("s") * NC + lax.axis_index("c")
    base = jnp.minimum(wid * ROWS_W, N - ROWS_W)
    base = pl.multiple_of(base, 8)

    def gather(item, b):
        c, w = item
        sz = C if c < FULL_CHUNKS else TAIL
        rows = pl.ds(base + c * C, sz)
        src = (s_hbm, d_hbm)[w]
        return pltpu.make_async_copy(
            src.at[rows, :], buf.at[b, pl.ds(0, sz), :], g_sem.at[b]
        )

    def scatter(item, b):
        c, w = item
        sz = C if c < FULL_CHUNKS else TAIL
        rows = pl.ds(base + c * C, sz)
        return pltpu.make_async_copy(
            buf.at[b, pl.ds(0, sz), :], o_hbm.at[rows, pl.ds(w * D, D)], s_sem.at[b]
        )

    n = len(_ITEMS)
    for i in range(n + 1):
        b = i % 2
        pb = (i - 1) % 2
        if i < n:
            if i >= 2:
                scatter(_ITEMS[i - 2], b).wait()
            gather(_ITEMS[i], b).start()
        if i >= 1:
            gather(_ITEMS[i - 1], pb).wait()
            scatter(_ITEMS[i - 1], pb).start()
    for i in (n - 2, n - 1):
        scatter(_ITEMS[i], i % 2).wait()


def kernel(static_emb, dynamic_emb):
    return _sc_concat(static_emb, dynamic_emb)


# speedup vs baseline: 1.0815x; 1.0701x over previous
import functools
import jax
import jax.numpy as jnp
from jax import lax
from jax.experimental import pallas as pl
from jax.experimental.pallas import tpu as pltpu
from jax.experimental.pallas import tpu_sc as plsc

N = 100000
D = 256
NC = 2
NS = 16
NW = NC * NS
ROWS_W = -(-(N // 8) // NW) * 8
C = 240
FULL_CHUNKS = ROWS_W // C
TAIL = ROWS_W - FULL_CHUNKS * C
NCH = FULL_CHUNKS + (1 if TAIL else 0)

_ITEMS = [(c, w) for c in range(NCH) for w in (0, 1)]

_mesh = plsc.VectorSubcoreMesh(core_axis_name="c", subcore_axis_name="s")


@functools.partial(
    pl.kernel,
    out_type=jax.ShapeDtypeStruct((N, 2 * D), jnp.float32),
    mesh=_mesh,
    scratch_types=[
        pltpu.VMEM_SHARED((NS, 2, C, D), jnp.float32),
        pltpu.SemaphoreType.DMA((2,)),
        pltpu.SemaphoreType.DMA((2,)),
    ],
)
def _sc_concat(s_hbm, d_hbm, o_hbm, sbuf, g_sem, s_sem):
    wid = lax.axis_index("s") * NC + lax.axis_index("c")
    buf = sbuf.at[lax.axis_index("s")]
    base = jnp.minimum(wid * ROWS_W, N - ROWS_W)
    base = pl.multiple_of(base, 8)

    def gather(item, b):
        c, w = item
        sz = C if c < FULL_CHUNKS else TAIL
        rows = pl.ds(base + c * C, sz)
        src = (s_hbm, d_hbm)[w]
        return pltpu.make_async_copy(
            src.at[rows, :], buf.at[b, pl.ds(0, sz), :], g_sem.at[b]
        )

    def scatter(item, b):
        c, w = item
        sz = C if c < FULL_CHUNKS else TAIL
        rows = pl.ds(base + c * C, sz)
        return pltpu.make_async_copy(
            buf.at[b, pl.ds(0, sz), :], o_hbm.at[rows, pl.ds(w * D, D)], s_sem.at[b]
        )

    n = len(_ITEMS)
    for i in range(n + 1):
        b = i % 2
        pb = (i - 1) % 2
        if i < n:
            if i >= 2:
                scatter(_ITEMS[i - 2], b).wait()
            gather(_ITEMS[i], b).start()
        if i >= 1:
            gather(_ITEMS[i - 1], pb).wait()
            scatter(_ITEMS[i - 1], pb).start()
    for i in (n - 2, n - 1):
        scatter(_ITEMS[i], i % 2).wait()


def kernel(static_emb, dynamic_emb):
    return _sc_concat(static_emb, dynamic_emb)
